# baseline (speedup 1.0000x reference)
import jax
import jax.numpy as jnp
from jax.experimental import pallas as pl

N_TOKENS = 16384
IN_FEATURES = 4096
N_EXPERTS = 64
TOP_K = 8
BM = 512
N_STREAMS = 2
HALF_BLOCKS = N_TOKENS // N_STREAMS // BM


def _topk_from_scores(s, w_out_ref, i_out_ref):
    rev_iota = jax.lax.broadcasted_iota(jnp.int32, s.shape, 1) ^ 63
    s_bits = jax.lax.bitcast_convert_type(s, jnp.int32)
    key = (s_bits & ~jnp.int32(63)) | rev_iota
    for j in range(TOP_K):
        cur = jnp.max(key, axis=1, keepdims=True)
        w_out_ref[:, j : j + 1] = jax.lax.bitcast_convert_type(
            cur & ~jnp.int32(63), jnp.float32
        )
        i_out_ref[:, j : j + 1] = (cur & 63) ^ 63
        key = jnp.where(key == cur, jnp.int32(-1), key)


def _gate_body(x_ref, wt_ref, w_out_ref, i_out_ref):
    logits = jnp.dot(x_ref[...], wt_ref[...], preferred_element_type=jnp.float32)
    m = jnp.max(logits, axis=1, keepdims=True)
    e = jnp.exp(logits - m)
    s = e / jnp.sum(e, axis=1, keepdims=True)
    _topk_from_scores(s, w_out_ref, i_out_ref)


def _gate_kernel(x0_ref, x1_ref, wt_ref, w0_ref, i0_ref, w1_ref, i1_ref):
    _gate_body(x0_ref, wt_ref, w0_ref, i0_ref)
    _gate_body(x1_ref, wt_ref, w1_ref, i1_ref)


def kernel(x, W):
    wt = W.T
    grid = (HALF_BLOCKS,)
    w0, i0, w1, i1 = pl.pallas_call(
        _gate_kernel,
        grid=grid,
        in_specs=[
            pl.BlockSpec((BM, IN_FEATURES), lambda i: (i, 0)),
            pl.BlockSpec((BM, IN_FEATURES), lambda i: (HALF_BLOCKS + i, 0)),
            pl.BlockSpec((IN_FEATURES, N_EXPERTS), lambda i: (0, 0)),
        ],
        out_specs=[
            pl.BlockSpec((BM, TOP_K), lambda i: (i, 0)),
            pl.BlockSpec((BM, TOP_K), lambda i: (i, 0)),
            pl.BlockSpec((BM, TOP_K), lambda i: (i, 0)),
            pl.BlockSpec((BM, TOP_K), lambda i: (i, 0)),
        ],
        out_shape=[
            jax.ShapeDtypeStruct((N_TOKENS // 2, TOP_K), jnp.float32),
            jax.ShapeDtypeStruct((N_TOKENS // 2, TOP_K), jnp.int32),
            jax.ShapeDtypeStruct((N_TOKENS // 2, TOP_K), jnp.float32),
            jax.ShapeDtypeStruct((N_TOKENS // 2, TOP_K), jnp.int32),
        ],
    )(x, x, wt)
    weights = jnp.concatenate([w0, w1], axis=0)
    indices = jnp.concatenate([i0, i1], axis=0)
    return weights, indices
